# baseline (speedup 1.0000x reference)
import functools

import jax
import jax.numpy as jnp
from jax import lax
from jax.experimental import pallas as pl
from jax.experimental.pallas import tpu as pltpu
from jax.experimental.pallas import tpu_sc as plsc


def _edge_sums_tc(xf):
    Bm, R, C = xf.shape
    W = C // 3

    def shifted(X, k, w):
        same = jnp.roll(X, -k, axis=1)
        wrap = jnp.roll(same, -1, axis=0)
        lane = jax.lax.broadcasted_iota(jnp.int32, (R, w), 1)
        return jnp.where(lane < w - k, same, wrap)

    def body(x_ref, s_ref):
        X = x_ref[0]
        d3 = shifted(X, 3, C) - X
        d6 = shifted(X, 6, C) - X
        sq3 = d3 * d3
        sq6 = d6 * d6
        j = jax.lax.broadcasted_iota(jnp.int32, (C, W), 0)
        c = jax.lax.broadcasted_iota(jnp.int32, (C, W), 1)
        sel = (j // 3 == c).astype(jnp.float32)
        e2 = jax.lax.dot_general(sq3, sel, (((1,), (0,)), ((), ())),
                                 preferred_element_type=jnp.float32)
        g2 = jax.lax.dot_general(sq6, sel, (((1,), (0,)), ((), ())),
                                 preferred_element_type=jnp.float32)
        e = jnp.sqrt(e2)
        g = jnp.sqrt(g2)
        s_ref[0] = e + shifted(e, 1, W) + g

    return pl.pallas_call(
        body,
        grid=(Bm,),
        in_specs=[pl.BlockSpec((1, R, C), lambda i: (i, 0, 0))],
        out_specs=pl.BlockSpec((1, R, W), lambda i: (i, 0, 0)),
        out_shape=jax.ShapeDtypeStruct((Bm, R, W), jnp.float32),
    )(xf)


def _gather_mean_sc(s_flat, faces_flat, Bm, V, F):
    try:
        info = plsc.get_sparse_core_info()
        NC, NS, L = info.num_cores, info.num_subcores, info.num_lanes
    except Exception:
        NC, NS, L = 2, 16, 16
    NW = NC * NS
    assert NW % Bm == 0
    TPM = NW // Bm
    assert TPM <= NS
    FC = F // TPM
    assert FC * TPM == F
    full_vregs = FC // L
    tail = FC - full_vregs * L
    assert (FC * 3) % 8 == 0 and (V * Bm) % 8 == 0
    inv_f = jnp.float32(1.0 / F)

    mesh = plsc.VectorSubcoreMesh(core_axis_name="c", subcore_axis_name="s")

    @functools.partial(
        pl.kernel,
        mesh=mesh,
        out_type=jax.ShapeDtypeStruct((Bm, L), jnp.float32),
        compiler_params=pltpu.CompilerParams(needs_layout_passes=False),
        scratch_types=[
            pltpu.VMEM((V,), jnp.float32),
            pltpu.VMEM((FC * 3,), jnp.int32),
            pltpu.VMEM((L,), jnp.float32),
            pltpu.VMEM((TPM, L), jnp.float32),
            pltpu.VMEM((L,), jnp.float32),
            pltpu.VMEM_SHARED((NS, L), jnp.float32),
        ],
    )
    def k(s_hbm, faces_hbm, out_hbm, s_v, f_v, acc_v, tmp_v, out_v, shared):
        cid = lax.axis_index("c")
        sid = lax.axis_index("s")
        b = cid * (NS // TPM) + sid // TPM
        chunk = sid % TPM

        pltpu.sync_copy(s_hbm.at[pl.ds(b * V, V)], s_v)
        fstart = b * (F * 3) + chunk * (FC * 3)
        pltpu.sync_copy(faces_hbm.at[pl.ds(fstart, FC * 3)], f_v)

        lane = lax.iota(jnp.int32, L)
        lane3 = lane * 3

        def body(j, acc):
            widx = lane3 + j * (3 * L)
            basev = plsc.load_gather(f_v, [widx])
            sval = plsc.load_gather(s_v, [basev])
            return acc + sval

        acc = lax.fori_loop(0, full_vregs, body,
                            jnp.zeros((L,), jnp.float32), unroll=8)

        if tail:
            widx = jnp.minimum(lane3 + full_vregs * (3 * L),
                               jnp.int32(FC * 3 - 3))
            basev = plsc.load_gather(f_v, [widx])
            sval = plsc.load_gather(s_v, [basev])
            acc = acc + jnp.where(lane < tail, sval, jnp.float32(0.0))

        acc_v[...] = acc
        pltpu.sync_copy(acc_v, shared.at[sid])
        plsc.subcore_barrier()

        @pl.when(chunk == 0)
        def _finalize():
            pltpu.sync_copy(shared.at[pl.ds(sid, TPM)], tmp_v)
            tot = tmp_v[0]
            for t in range(1, TPM):
                tot = tot + tmp_v[t]
            total = jnp.sum(tot)
            out_v[...] = jnp.full((L,), total * inv_f, jnp.float32)
            pltpu.sync_copy(out_v, out_hbm.at[b])

    return k(s_flat, faces_flat)


def kernel(vertices_batch, faces_batch):
    Bm, V, _ = vertices_batch.shape
    _, F, _ = faces_batch.shape
    faces_flat = faces_batch.astype(jnp.int32).reshape(-1)
    C = 240
    R = (3 * V) // C
    assert R * C == 3 * V
    xf = vertices_batch.reshape(Bm, R, C)
    s = _edge_sums_tc(xf)
    out = _gather_mean_sc(s.reshape(-1), faces_flat, Bm, V, F)
    return out[:, 0]


# speedup vs baseline: 74.9970x; 74.9970x over previous
import functools

import jax
import jax.numpy as jnp
from jax import lax
from jax.experimental import pallas as pl
from jax.experimental.pallas import tpu as pltpu
from jax.experimental.pallas import tpu_sc as plsc


def _dense_tc(verts24, faces24, Bm, V, F):
    TS = (V + 127) // 128
    TF = (F + 127) // 128
    TF4 = ((TF + 3) // 4) * 4

    def body(x_ref, f_ref, s_ref, b_ref):
        X = x_ref[...]
        r1 = jnp.roll(X, -1, axis=1)
        r2 = jnp.roll(X, -2, axis=1)
        d1 = r1 - X
        d2 = r2 - X
        q1 = d1 * d1
        q2 = d2 * d2
        e2 = q1[0:Bm] + q1[Bm:2 * Bm] + q1[2 * Bm:3 * Bm]
        g2 = q2[0:Bm] + q2[Bm:2 * Bm] + q2[2 * Bm:3 * Bm]
        e = jnp.sqrt(e2)
        g = jnp.sqrt(g2)
        s = e + jnp.roll(e, -1, axis=1) + g
        for t in range(TS - 1):
            s_ref[t] = s[:, 128 * t:128 * (t + 1)]
        w = V - 128 * (TS - 1)
        s_ref[TS - 1, :, 0:w] = s[:, 128 * (TS - 1):V]
        Fb = f_ref[...]
        for t in range(TF - 1):
            b_ref[t] = Fb[:, 128 * t:128 * (t + 1)]
        wf = F - 128 * (TF - 1)
        b_ref[TF - 1, :, 0:wf] = Fb[:, 128 * (TF - 1):F]

    return pl.pallas_call(
        body,
        grid=(1,),
        in_specs=[
            pl.BlockSpec((3 * Bm, V), lambda i: (0, 0)),
            pl.BlockSpec((Bm, F), lambda i: (0, 0)),
        ],
        out_specs=[
            pl.BlockSpec((TS, Bm, 128), lambda i: (0, 0, 0)),
            pl.BlockSpec((TF4, Bm, 128), lambda i: (0, 0, 0)),
        ],
        out_shape=[
            jax.ShapeDtypeStruct((TS, Bm, 128), jnp.float32),
            jax.ShapeDtypeStruct((TF4, Bm, 128), jnp.int32),
        ],
    )(verts24, faces24)


def _gather_mean_sc(s3d, base3d, Bm, V, F):
    try:
        info = plsc.get_sparse_core_info()
        NC, NS, L = info.num_cores, info.num_subcores, info.num_lanes
    except Exception:
        NC, NS, L = 2, 16, 16
    NW = NC * NS
    assert NW % Bm == 0
    TPM = NW // Bm
    TS = s3d.shape[0]
    TF4 = base3d.shape[0]
    TC_ = TF4 // TPM
    assert TC_ * TPM == TF4
    valids = [min((c + 1) * TC_ * 128, F) - c * TC_ * 128 for c in range(TPM)]
    assert all(v % L == 0 for v in valids)
    nv_min = min(valids) // L
    inv_f = jnp.float32(1.0 / F)

    mesh = plsc.VectorSubcoreMesh(core_axis_name="c", subcore_axis_name="s")

    @functools.partial(
        pl.kernel,
        mesh=mesh,
        out_type=jax.ShapeDtypeStruct((Bm, L), jnp.float32),
        compiler_params=pltpu.CompilerParams(needs_layout_passes=False),
        scratch_types=[
            pltpu.VMEM((TS, 128), jnp.float32),
            pltpu.VMEM((TC_, 128), jnp.int32),
            pltpu.VMEM((L,), jnp.float32),
            pltpu.VMEM((TPM, L), jnp.float32),
            pltpu.VMEM((L,), jnp.float32),
            pltpu.VMEM_SHARED((NS, L), jnp.float32),
        ],
    )
    def k(s_hbm, b_hbm, out_hbm, s_v, f_v, acc_v, tmp_v, out_v, shared):
        cid = lax.axis_index("c")
        sid = lax.axis_index("s")
        b = cid * (NS // TPM) + sid // TPM
        chunk = sid % TPM

        pltpu.sync_copy(s_hbm.at[:, b, :], s_v)
        pltpu.sync_copy(b_hbm.at[pl.ds(chunk * TC_, TC_), b, :], f_v)

        start_w = chunk * (TC_ * 128)
        nv = (jnp.minimum(start_w + TC_ * 128, F) - start_w) // L

        def body(j, acc):
            t = j >> 3
            off = (j & 7) * L
            basev = f_v[t, pl.ds(off, L)]
            sval = plsc.load_gather(s_v, [basev >> 7, basev & 127])
            return acc + sval

        acc = lax.fori_loop(0, nv_min, body,
                            jnp.zeros((L,), jnp.float32), unroll=8)
        acc = lax.fori_loop(nv_min, nv, body, acc)

        acc_v[...] = acc
        pltpu.sync_copy(acc_v, shared.at[sid])
        plsc.subcore_barrier()

        @pl.when(chunk == 0)
        def _finalize():
            pltpu.sync_copy(shared.at[pl.ds(sid, TPM)], tmp_v)
            tot = tmp_v[0]
            for t in range(1, TPM):
                tot = tot + tmp_v[t]
            total = jnp.sum(tot)
            out_v[...] = jnp.full((L,), total * inv_f, jnp.float32)
            pltpu.sync_copy(out_v, out_hbm.at[b])

    return k(s3d, base3d)


def kernel(vertices_batch, faces_batch):
    Bm, V, _ = vertices_batch.shape
    _, F, _ = faces_batch.shape
    verts24 = jnp.transpose(vertices_batch, (2, 0, 1)).reshape(3 * Bm, V)
    faces24 = jnp.transpose(faces_batch.astype(jnp.int32),
                            (2, 0, 1)).reshape(3 * Bm, F)
    s3d, base3d = _dense_tc(verts24, faces24, Bm, V, F)
    out = _gather_mean_sc(s3d, base3d, Bm, V, F)
    return out[:, 0]


# speedup vs baseline: 75.2054x; 1.0028x over previous
import functools

import jax
import jax.numpy as jnp
from jax import lax
from jax.experimental import pallas as pl
from jax.experimental.pallas import tpu as pltpu
from jax.experimental.pallas import tpu_sc as plsc


def _dense_tc(verts24, faces24, Bm, V, F):
    TS = (V + 127) // 128
    TF = (F + 127) // 128
    TF4 = ((TF + 3) // 4) * 4

    def body(x_ref, f_ref, s_ref, b_ref):
        X = x_ref[...]
        r1 = jnp.roll(X, -1, axis=1)
        r2 = jnp.roll(X, -2, axis=1)
        d1 = r1 - X
        d2 = r2 - X
        q1 = d1 * d1
        q2 = d2 * d2
        e2 = q1[0:Bm] + q1[Bm:2 * Bm] + q1[2 * Bm:3 * Bm]
        g2 = q2[0:Bm] + q2[Bm:2 * Bm] + q2[2 * Bm:3 * Bm]
        e = jnp.sqrt(e2)
        g = jnp.sqrt(g2)
        s = e + jnp.roll(e, -1, axis=1) + g
        for t in range(TS - 1):
            s_ref[t] = s[:, 128 * t:128 * (t + 1)]
        w = V - 128 * (TS - 1)
        s_ref[TS - 1, :, 0:w] = s[:, 128 * (TS - 1):V]
        Fb = f_ref[...]
        for t in range(TF - 1):
            b_ref[t] = Fb[:, 128 * t:128 * (t + 1)]
        wf = F - 128 * (TF - 1)
        b_ref[TF - 1, :, 0:wf] = Fb[:, 128 * (TF - 1):F]

    return pl.pallas_call(
        body,
        grid=(1,),
        in_specs=[
            pl.BlockSpec((3 * Bm, V), lambda i: (0, 0)),
            pl.BlockSpec((Bm, F), lambda i: (0, 0)),
        ],
        out_specs=[
            pl.BlockSpec((TS, Bm, 128), lambda i: (0, 0, 0)),
            pl.BlockSpec((TF4, Bm, 128), lambda i: (0, 0, 0)),
        ],
        out_shape=[
            jax.ShapeDtypeStruct((TS, Bm, 128), jnp.float32),
            jax.ShapeDtypeStruct((TF4, Bm, 128), jnp.int32),
        ],
    )(verts24, faces24)


def _gather_mean_sc(s3d, base3d, Bm, V, F):
    try:
        info = plsc.get_sparse_core_info()
        NC, NS, L = info.num_cores, info.num_subcores, info.num_lanes
    except Exception:
        NC, NS, L = 2, 16, 16
    NW = NC * NS
    assert NW % Bm == 0
    TPM = NW // Bm
    TS = s3d.shape[0]
    TF4 = base3d.shape[0]
    TC_ = TF4 // TPM
    assert TC_ * TPM == TF4
    valids = [min((c + 1) * TC_ * 128, F) - c * TC_ * 128 for c in range(TPM)]
    assert all(v % L == 0 for v in valids)
    nv_min = min(valids) // L
    inv_f = jnp.float32(1.0 / F)

    mesh = plsc.VectorSubcoreMesh(core_axis_name="c", subcore_axis_name="s")

    @functools.partial(
        pl.kernel,
        mesh=mesh,
        out_type=jax.ShapeDtypeStruct((Bm, L), jnp.float32),
        compiler_params=pltpu.CompilerParams(needs_layout_passes=False),
        scratch_types=[
            pltpu.VMEM((TS, 128), jnp.float32),
            pltpu.VMEM((TC_, 128), jnp.int32),
            pltpu.VMEM((L,), jnp.float32),
            pltpu.VMEM((TPM, L), jnp.float32),
            pltpu.VMEM((L,), jnp.float32),
            pltpu.VMEM_SHARED((NS, L), jnp.float32),
        ],
    )
    def k(s_hbm, b_hbm, out_hbm, s_v, f_v, acc_v, tmp_v, out_v, shared):
        cid = lax.axis_index("c")
        sid = lax.axis_index("s")
        b = cid * (NS // TPM) + sid // TPM
        chunk = sid % TPM

        pltpu.sync_copy(s_hbm.at[:, b, :], s_v)
        pltpu.sync_copy(b_hbm.at[pl.ds(chunk * TC_, TC_), b, :], f_v)

        start_w = chunk * (TC_ * 128)
        nv = (jnp.minimum(start_w + TC_ * 128, F) - start_w) // L

        def body(j, acc):
            t = j >> 3
            off = (j & 7) * L
            basev = f_v[t, pl.ds(off, L)]
            sval = plsc.load_gather(s_v, [basev >> 7, basev & 127])
            return acc + sval

        acc = lax.fori_loop(0, nv_min, body,
                            jnp.zeros((L,), jnp.float32), unroll=8)
        acc = lax.fori_loop(nv_min, nv, body, acc)

        acc_v[...] = acc
        pltpu.sync_copy(acc_v, shared.at[sid])
        plsc.subcore_barrier()

        @pl.when(chunk == 0)
        def _finalize():
            pltpu.sync_copy(shared.at[pl.ds(sid, TPM)], tmp_v)
            tot = tmp_v[0]
            for t in range(1, TPM):
                tot = tot + tmp_v[t]
            total = jnp.sum(tot)
            out_v[...] = jnp.full((L,), total * inv_f, jnp.float32)
            pltpu.sync_copy(out_v, out_hbm.at[b])

    return k(s3d, base3d)


def kernel(vertices_batch, faces_batch):
    Bm, V, _ = vertices_batch.shape
    _, F, _ = faces_batch.shape
    verts24 = jnp.transpose(vertices_batch, (2, 0, 1)).reshape(3 * Bm, V)
    faces24 = jnp.transpose(faces_batch.astype(jnp.int32),
                            (2, 0, 1)).reshape(3 * Bm, F)
    verts24 = pltpu.with_memory_space_constraint(verts24, pltpu.MemorySpace.HBM)
    faces24 = pltpu.with_memory_space_constraint(faces24, pltpu.MemorySpace.HBM)
    s3d, base3d = _dense_tc(verts24, faces24, Bm, V, F)
    s3d = pltpu.with_memory_space_constraint(s3d, pltpu.MemorySpace.HBM)
    base3d = pltpu.with_memory_space_constraint(base3d, pltpu.MemorySpace.HBM)
    out = _gather_mean_sc(s3d, base3d, Bm, V, F)
    return out[:, 0]
